# baseline (speedup 1.0000x reference)
import functools

import jax
import jax.numpy as jnp
from jax import lax
from jax.experimental import pallas as pl
from jax.experimental.pallas import tpu as pltpu
from jax.experimental.pallas import tpu_sc as plsc

B = 16384
D = 32
V = 1000000
FV = V // 4
NC, NS, L = 2, 16, 16
NW = NC * NS
BPW = B // NW
CHUNK = 128
NCH = BPW // CHUNK
CR = 4000
CSTEPS = V // CR


def _tc_repack(user_embedding, item_embedding):
    def body(u_ref, i_ref, fu_ref, fi_ref):
        def wide(x):
            return jnp.concatenate([x, x, x, x], axis=1)

        fu_ref[...] = wide(u_ref[...])
        fi_ref[...] = wide(i_ref[...])

    return pl.pallas_call(
        body,
        grid=(CSTEPS,),
        in_specs=[
            pl.BlockSpec((CR, D), lambda g: (g, 0)),
            pl.BlockSpec((CR, D), lambda g: (g, 0)),
        ],
        out_specs=[
            pl.BlockSpec((CR, 128), lambda g: (g, 0)),
            pl.BlockSpec((CR, 128), lambda g: (g, 0)),
        ],
        out_shape=[jax.ShapeDtypeStruct((V, 128), jnp.float32),
                   jax.ShapeDtypeStruct((V, 128), jnp.float32)],
    )(user_embedding, item_embedding)


def _sc_gather(ft_u, ft_i, u, i, j):
    mesh = plsc.VectorSubcoreMesh(core_axis_name="c", subcore_axis_name="s")

    @functools.partial(
        pl.kernel,
        mesh=mesh,
        out_type=[
            jax.ShapeDtypeStruct((B, 128), jnp.float32),
            jax.ShapeDtypeStruct((B, 128), jnp.float32),
            jax.ShapeDtypeStruct((B, 128), jnp.float32),
        ],
        scratch_types=[
            pltpu.VMEM((NCH, CHUNK), jnp.int32),
            pltpu.VMEM((BPW, 128), jnp.float32),
            pltpu.SemaphoreType.DMA,
        ],
    )
    def k(tu_hbm, ti_hbm, u_hbm, i_hbm, j_hbm, gu_hbm, gp_hbm, gn_hbm,
          q_s, d_s, sem):
        wid = lax.axis_index("s") * NC + lax.axis_index("c")
        base = wid * BPW

        def gather(idx_hbm, table, out_hbm):
            pltpu.sync_copy(idx_hbm.at[pl.ds(wid * NCH, NCH)], q_s)

            copies = []
            for ch in range(NCH):
                copies.append(pltpu.async_copy(
                    table.at[q_s.at[ch]],
                    d_s.at[pl.ds(ch * CHUNK, CHUNK)], sem))
            for cp in copies:
                cp.wait()
            pltpu.sync_copy(d_s, out_hbm.at[pl.ds(base, BPW)])

        gather(u_hbm, tu_hbm, gu_hbm)
        gather(i_hbm, ti_hbm, gp_hbm)
        gather(j_hbm, ti_hbm, gn_hbm)

    return k(ft_u, ft_i, u, i, j)


def _tc_reduce(gu, gp, gn):
    def body(u_ref, p_ref, n_ref, bpr_ref, reg_ref):
        lane = lax.broadcasted_iota(jnp.int32, (B, 128), 1)
        mask = lane < D
        u = u_ref[...]
        p = p_ref[...]
        n = n_ref[...]
        h = jnp.where(mask, u * (n - p), 0.0)
        score = jnp.sum(h, axis=1)
        sp = jnp.maximum(score, 0.0) + jnp.log(1.0 + jnp.exp(-jnp.abs(score)))
        bpr = jnp.sum(sp)
        squ = jnp.where(mask, u * u + p * p + n * n, 0.0)
        reg = jnp.sum(squ) * (1.0 / B)
        bpr_ref[...] = jnp.full((8, 128), bpr, jnp.float32)
        reg_ref[...] = jnp.full((8, 128), reg, jnp.float32)

    bpr, reg = pl.pallas_call(
        body,
        out_shape=[jax.ShapeDtypeStruct((8, 128), jnp.float32),
                   jax.ShapeDtypeStruct((8, 128), jnp.float32)],
    )(gu, gp, gn)
    return bpr[0, 0], reg[0, 0]


def kernel(user_embedding, item_embedding, u, i, j):
    u = u.astype(jnp.int32)
    i = i.astype(jnp.int32)
    j = j.astype(jnp.int32)
    ft_u, ft_i = _tc_repack(user_embedding, item_embedding)
    gu, gp, gn = _sc_gather(ft_u, ft_i, u.reshape(NW * NCH, CHUNK),
                            i.reshape(NW * NCH, CHUNK),
                            j.reshape(NW * NCH, CHUNK))
    return _tc_reduce(gu, gp, gn)


# speedup vs baseline: 2.3817x; 2.3817x over previous
import functools

import jax
import jax.numpy as jnp
from jax import lax
from jax.experimental import pallas as pl
from jax.experimental.pallas import tpu as pltpu
from jax.experimental.pallas import tpu_sc as plsc

B = 16384
D = 32
NC, NS, L = 2, 16, 16
NW = NC * NS
BPW = B // NW
OR = B // 4


def _sc_gather(user_embedding, item_embedding, u, i, j):
    mesh = plsc.VectorSubcoreMesh(core_axis_name="c", subcore_axis_name="s")

    @functools.partial(
        pl.kernel,
        mesh=mesh,
        out_type=[
            jax.ShapeDtypeStruct((B, D), jnp.float32),
            jax.ShapeDtypeStruct((B, D), jnp.float32),
            jax.ShapeDtypeStruct((B, D), jnp.float32),
        ],
        scratch_types=[
            pltpu.VMEM((BPW,), jnp.int32),
            pltpu.VMEM((BPW, D), jnp.float32),
            pltpu.SemaphoreType.DMA,
        ],
    )
    def k(tu_hbm, ti_hbm, u_hbm, i_hbm, j_hbm, gu_hbm, gp_hbm, gn_hbm,
          idx_s, stage, sem):
        wid = lax.axis_index("s") * NC + lax.axis_index("c")
        base = wid * BPW

        def gather(idx_hbm, table, out_hbm):
            pltpu.sync_copy(idx_hbm.at[pl.ds(base, BPW)], idx_s)

            def body(g, carry):
                v = idx_s[pl.ds(g * L, L)]
                for r in range(L):
                    idx = v[r]
                    pos = g * L + r
                    pltpu.async_copy(
                        table.at[pl.ds(idx, 1), :],
                        stage.at[pl.ds(pos, 1), :],
                        sem)
                return carry

            lax.fori_loop(0, BPW // L, body, 0)
            pltpu.make_async_copy(
                table.at[pl.ds(0, BPW), :], stage, sem).wait()
            pltpu.sync_copy(stage, out_hbm.at[pl.ds(base, BPW), :])

        gather(u_hbm, tu_hbm, gu_hbm)
        gather(i_hbm, ti_hbm, gp_hbm)
        gather(j_hbm, ti_hbm, gn_hbm)

    return k(user_embedding, item_embedding, u, i, j)


def _tc_reduce(gu, gp, gn):
    def body(u_ref, p_ref, n_ref, bpr_ref, reg_ref):
        un = u_ref[...]
        pn = p_ref[...]
        nn = n_ref[...]
        h = jnp.sum(un * (nn - pn), axis=1)
        sp = jnp.maximum(h, 0.0) + jnp.log(1.0 + jnp.exp(-jnp.abs(h)))
        bpr = jnp.sum(sp)
        reg = jnp.sum(un * un + pn * pn + nn * nn) * (1.0 / B)
        bpr_ref[...] = jnp.full((8, 128), bpr, jnp.float32)
        reg_ref[...] = jnp.full((8, 128), reg, jnp.float32)

    bpr, reg = pl.pallas_call(
        body,
        out_shape=[jax.ShapeDtypeStruct((8, 128), jnp.float32),
                   jax.ShapeDtypeStruct((8, 128), jnp.float32)],
    )(gu, gp, gn)
    return bpr[0, 0], reg[0, 0]


def kernel(user_embedding, item_embedding, u, i, j):
    u = u.astype(jnp.int32)
    i = i.astype(jnp.int32)
    j = j.astype(jnp.int32)
    gu, gp, gn = _sc_gather(user_embedding, item_embedding, u, i, j)
    return _tc_reduce(gu, gp, gn)
